# baseline (speedup 1.0000x reference)
import functools

import jax
import jax.numpy as jnp
from jax import lax
from jax.experimental import pallas as pl
from jax.experimental.pallas import tpu as pltpu
from jax.experimental.pallas import tpu_sc as plsc

_MU = 0.02
_BINS = 10
_LOSS_WEIGHT = 1.0

_L = 16
_NC = 2
_NS = 16
_NW = _NC * _NS
_BINS_PAD = 16
_CHUNK = 8192


def _make_hist_body(rows, chunk_rows):
    nchunks = rows // chunk_rows
    tail_rows = rows - nchunks * chunk_rows
    steps = (nchunks + _NW - 1) // _NW

    def body(pred_2d, target_2d, lw_2d, cnt_out, sum_out,
             bufs, hcnt, hsum, sems):
        wid = lax.axis_index("s") * _NC + lax.axis_index("c")

        lane = lax.iota(jnp.int32, _L)
        zeros = jnp.zeros((_L,), jnp.float32)
        for r in range(_BINS_PAD):
            hcnt[pl.ds(r * _L, _L)] = zeros
            hsum[pl.ds(r * _L, _L)] = zeros

        srcs = (pred_2d, target_2d, lw_2d)
        ncol = pred_2d.shape[1]
        rpv = _L // ncol

        def copies(s, slot, nr):
            hrow0 = (wid + s * _NW) * chunk_rows * rpv
            return [
                pltpu.make_async_copy(
                    srcs[a].at[pl.ds(hrow0, nr * rpv), :],
                    bufs[slot][a].at[pl.ds(0, nr * rpv), :],
                    sems[slot][a])
                for a in range(3)
            ]

        mu2 = jnp.float32(_MU * _MU)
        rowpat = lane >> 2
        colpat = lane & (ncol - 1)

        def make_step(bp, bt, bw):
            def step(k, carry):
                rws = rowpat + k * rpv
                p = plsc.load_gather(bp, [rws, colpat])
                t = plsc.load_gather(bt, [rws, colpat])
                w = plsc.load_gather(bw, [rws, colpat])
                d = p - t
                x = d * d + mu2
                xi = lax.bitcast_convert_type(x, jnp.int32)
                yi = jnp.int32(0x5F3759DF) - (xi >> 1)
                y = lax.bitcast_convert_type(yi, jnp.float32)
                y = y * (1.5 - 0.5 * x * y * y)
                y = y * (1.5 - 0.5 * x * y * y)
                loss = x * y - _MU
                g10 = jnp.abs(d) * y * 10.0
                bini = jnp.minimum(g10.astype(jnp.int32), _BINS - 1)
                slot_idx = bini * _L + lane
                validm = w > 0.0
                cntv = jnp.where(validm, 1.0, 0.0).astype(jnp.float32)
                lossv = jnp.where(validm, loss, 0.0)
                plsc.addupdate_scatter(hcnt, [slot_idx], cntv)
                plsc.addupdate_scatter(hsum, [slot_idx], lossv)
                return carry

            return step

        def has_chunk(s):
            return wid + s * _NW < nchunks

        @pl.when(has_chunk(0))
        def _():
            for c in copies(0, 0, chunk_rows):
                c.start()

        for s in range(steps):
            slot = s % 2
            if s + 1 < steps:
                @pl.when(has_chunk(s + 1))
                def _():
                    for c in copies(s + 1, 1 - slot, chunk_rows):
                        c.start()

            @pl.when(has_chunk(s))
            def _():
                for c in copies(s, slot, chunk_rows):
                    c.wait()
                bp, bt, bw = bufs[slot]
                lax.fori_loop(0, chunk_rows, make_step(bp, bt, bw), 0,
                              unroll=8)

        if tail_rows:
            @pl.when(wid == _NW - 1)
            def _():
                hrow0 = nchunks * chunk_rows * rpv
                for a in range(3):
                    pltpu.sync_copy(
                        srcs[a].at[pl.ds(hrow0, tail_rows * rpv), :],
                        bufs[0][a].at[pl.ds(0, tail_rows * rpv), :])
                bp, bt, bw = bufs[0]
                lax.fori_loop(0, tail_rows, make_step(bp, bt, bw), 0,
                              unroll=8)

        pltpu.sync_copy(hcnt, cnt_out.at[wid])
        pltpu.sync_copy(hsum, sum_out.at[wid])

    return body


def _epilogue_body(cnt_ref, sum_ref, out_ref):
    c = cnt_ref[...]
    s = sum_ref[...]
    cb = jnp.sum(jnp.sum(c, axis=0), axis=1, keepdims=True)
    sb = jnp.sum(jnp.sum(s, axis=0), axis=1, keepdims=True)
    nz = cb > 0.0
    n = jnp.sum(nz.astype(jnp.float32))
    denom = jnp.where(nz, cb * n, 1.0)
    contrib = jnp.where(nz, sb / denom, 0.0)
    total = jnp.sum(contrib, keepdims=True) * jnp.float32(_LOSS_WEIGHT)
    out_ref[...] = total.reshape(1, 1)


def kernel(pred, target, label_weight):
    total = pred.size
    assert total % _L == 0
    rows = total // _L
    chunk_rows = _CHUNK // _L

    mesh = plsc.VectorSubcoreMesh(core_axis_name="c", subcore_axis_name="s")
    hist = pl.kernel(
        _make_hist_body(rows, chunk_rows),
        out_type=(
            jax.ShapeDtypeStruct((_NW, _BINS_PAD * _L), jnp.float32),
            jax.ShapeDtypeStruct((_NW, _BINS_PAD * _L), jnp.float32),
        ),
        mesh=mesh,
        scratch_types=(
            tuple(tuple(pltpu.VMEM((chunk_rows * (_L // 4), 4), jnp.float32)
                        for _ in range(3))
                  for _ in range(2)),
            pltpu.VMEM((_BINS_PAD * _L,), jnp.float32),
            pltpu.VMEM((_BINS_PAD * _L,), jnp.float32),
            tuple(tuple(pltpu.SemaphoreType.DMA for _ in range(3))
                  for _ in range(2)),
        ),
        compiler_params=pltpu.CompilerParams(
            needs_layout_passes=False, use_tc_tiling_on_sc=False),
    )
    cnt, sums = hist(pred, target, label_weight)
    cnt = cnt.reshape(_NW, _BINS_PAD, _L)
    sums = sums.reshape(_NW, _BINS_PAD, _L)

    out = pl.pallas_call(
        _epilogue_body,
        out_shape=jax.ShapeDtypeStruct((1, 1), jnp.float32),
    )(cnt, sums)
    return out[0, 0]


# speedup vs baseline: 31.1320x; 31.1320x over previous
import jax
import jax.numpy as jnp
from jax import lax
from jax.experimental import pallas as pl
from jax.experimental.pallas import tpu as pltpu
from jax.experimental.pallas import tpu_sc as plsc

_MU = 0.02
_BINS = 10
_LOSS_WEIGHT = 1.0

_L = 16
_NC = 2
_NS = 16
_NW = _NC * _NS
_BINS_PAD = 16
_BW = 16384


def _make_elem_body(n_valid, bw):
    mu2 = _MU * _MU

    def body(p_ref, t_ref, w_ref, lv_ref, bn_ref):
        b = pl.program_id(0)
        p = p_ref[...]
        t = t_ref[...]
        w = w_ref[...]
        colg = b * bw + lax.broadcasted_iota(jnp.int32, p.shape, 1)
        d = p - t
        x = d * d + mu2
        loss = jnp.sqrt(x) - _MU
        g10 = jnp.abs(d) * lax.rsqrt(x) * 10.0
        bini = jnp.minimum(g10.astype(jnp.int32), _BINS - 1)
        ok = (w > 0.0) & (colg < n_valid)
        lv = jnp.where(ok, loss, 0.0)
        bn = jnp.where(ok, bini, _BINS_PAD - 1)
        rb = p.size // 128
        lv_ref[...] = lv.reshape(rb, 128)
        bn_ref[...] = bn.reshape(rb, 128)

    return body


def _make_hist_body(out_r):
    rows_w = out_r // _NW
    nchunks = 16
    cr = rows_w // nchunks

    def body(lv_hbm, bn_hbm, cnt_out, sum_out, bufs, hcnt, hsum, sems):
        wid = lax.axis_index("s") * _NC + lax.axis_index("c")
        base = wid * rows_w

        lane = lax.iota(jnp.int32, _L)
        ones = jnp.ones((_L,), jnp.float32)
        zeros = jnp.zeros((_L,), jnp.float32)
        for r in range(_BINS_PAD):
            hcnt[pl.ds(r * _L, _L)] = zeros
            hsum[pl.ds(r * _L, _L)] = zeros

        srcs = (lv_hbm, bn_hbm)

        def copies(j, slot):
            row0 = base + j * cr
            return [
                pltpu.make_async_copy(
                    srcs[a].at[pl.ds(row0, cr), :],
                    bufs[slot][a],
                    sems[slot][a])
                for a in range(2)
            ]

        prime = copies(0, 0)
        for c in prime:
            c.start()

        for j in range(nchunks):
            slot = j % 2
            if j + 1 < nchunks:
                for c in copies(j + 1, 1 - slot):
                    c.start()
            for c in copies(j, slot):
                c.wait()
            blv, bbn = bufs[slot]

            def step(r, carry):
                for c in range(128 // _L):
                    lvv = blv[r, pl.ds(c * _L, _L)]
                    bnv = bbn[r, pl.ds(c * _L, _L)]
                    slot_i = bnv * _L + lane
                    plsc.addupdate_scatter(hsum, [slot_i], lvv)
                    plsc.addupdate_scatter(hcnt, [slot_i], ones)
                return carry

            lax.fori_loop(0, cr, step, 0, unroll=2)

        pltpu.sync_copy(hcnt, cnt_out.at[wid])
        pltpu.sync_copy(hsum, sum_out.at[wid])

    return body


def _epilogue_body(cnt_ref, sum_ref, out_ref):
    c = cnt_ref[...]
    s = sum_ref[...]
    cb = jnp.sum(jnp.sum(c, axis=0), axis=1, keepdims=True)
    sb = jnp.sum(jnp.sum(s, axis=0), axis=1, keepdims=True)
    rowid = lax.broadcasted_iota(jnp.int32, cb.shape, 0)
    nz = (cb > 0.0) & (rowid < _BINS)
    n = jnp.sum(nz.astype(jnp.float32))
    denom = jnp.where(nz, cb * n, 1.0)
    contrib = jnp.where(nz, sb / denom, 0.0)
    total = jnp.sum(contrib, keepdims=True) * jnp.float32(_LOSS_WEIGHT)
    out_ref[...] = total.reshape(1, 1)


def kernel(pred, target, label_weight):
    n_rows, ncol = pred.shape
    nb = -(-n_rows // _BW)
    rb = ncol * _BW // 128
    out_r = nb * rb
    assert out_r % (_NW * 16) == 0

    xt = pred.T
    tt = target.T
    wt = label_weight.T

    lossv, binc = pl.pallas_call(
        _make_elem_body(n_rows, _BW),
        grid=(nb,),
        in_specs=[pl.BlockSpec((ncol, _BW), lambda b: (0, b))] * 3,
        out_specs=[pl.BlockSpec((rb, 128), lambda b: (b, 0))] * 2,
        out_shape=[
            jax.ShapeDtypeStruct((out_r, 128), jnp.float32),
            jax.ShapeDtypeStruct((out_r, 128), jnp.int32),
        ],
    )(xt, tt, wt)

    cr = out_r // _NW // 16
    mesh = plsc.VectorSubcoreMesh(core_axis_name="c", subcore_axis_name="s")
    hist = pl.kernel(
        _make_hist_body(out_r),
        out_type=(
            jax.ShapeDtypeStruct((_NW, _BINS_PAD * _L), jnp.float32),
            jax.ShapeDtypeStruct((_NW, _BINS_PAD * _L), jnp.float32),
        ),
        mesh=mesh,
        scratch_types=(
            tuple((pltpu.VMEM((cr, 128), jnp.float32),
                   pltpu.VMEM((cr, 128), jnp.int32))
                  for _ in range(2)),
            pltpu.VMEM((_BINS_PAD * _L,), jnp.float32),
            pltpu.VMEM((_BINS_PAD * _L,), jnp.float32),
            tuple(tuple(pltpu.SemaphoreType.DMA for _ in range(2))
                  for _ in range(2)),
        ),
        compiler_params=pltpu.CompilerParams(
            needs_layout_passes=False, use_tc_tiling_on_sc=False),
    )
    cnt, sums = hist(lossv, binc)
    cnt = cnt.reshape(_NW, _BINS_PAD, _L)
    sums = sums.reshape(_NW, _BINS_PAD, _L)

    out = pl.pallas_call(
        _epilogue_body,
        out_shape=jax.ShapeDtypeStruct((1, 1), jnp.float32),
    )(cnt, sums)
    return out[0, 0]
